# baseline (speedup 1.0000x reference)
import functools

import jax
import jax.numpy as jnp
from jax import lax
from jax.experimental import pallas as pl
from jax.experimental.pallas import tpu as pltpu
from jax.experimental.pallas import tpu_sc as plsc

_N = 100000
_D = 128
_H = 64
_S = 1024

_BLK = 20480
_NB16 = _N // 16
_WBLK = 391
_CHUNK = _WBLK * 16


def _mlp_body(h_ref, w0_ref, b0_ref, w1_ref, b1_ref, sc_ref, y_ref):
    wf = jnp.dot(w0_ref[...], w1_ref[...], preferred_element_type=jnp.float32)
    wf = wf * sc_ref[...]
    c = (jnp.dot(b0_ref[...], w1_ref[...], preferred_element_type=jnp.float32)
         + b1_ref[...]) * sc_ref[...]
    y = jnp.dot(h_ref[...], wf, preferred_element_type=jnp.float32) + c
    y_ref[...] = y.reshape(_BLK)


def _node_energies(h, W0, b0, W1, b1, scale):
    return pl.pallas_call(
        _mlp_body,
        grid=((_N + _BLK - 1) // _BLK,),
        in_specs=[
            pl.BlockSpec((_BLK, _D), lambda i: (i, 0)),
            pl.BlockSpec((_D, _H), lambda i: (0, 0)),
            pl.BlockSpec((1, _H), lambda i: (0, 0)),
            pl.BlockSpec((_H, 1), lambda i: (0, 0)),
            pl.BlockSpec((1, 1), lambda i: (0, 0)),
            pl.BlockSpec((1, 1), lambda i: (0, 0)),
        ],
        out_specs=pl.BlockSpec((_BLK,), lambda i: (i,)),
        out_shape=jax.ShapeDtypeStruct((_N,), jnp.float32),
    )(h, W0, b0.reshape(1, _H), W1, b1.reshape(1, 1), scale.reshape(1, 1))


def _make_segment_sum():
    mesh = plsc.VectorSubcoreMesh(
        core_axis_name="c", subcore_axis_name="s", num_cores=1)

    @functools.partial(
        pl.kernel,
        mesh=mesh,
        compiler_params=pltpu.CompilerParams(needs_layout_passes=False),
        out_type=jax.ShapeDtypeStruct((_S,), jnp.float32),
        scratch_types=[
            pltpu.VMEM((_CHUNK,), jnp.float32),
            pltpu.VMEM((_CHUNK,), jnp.int32),
            pltpu.VMEM((_S,), jnp.float32),
            pltpu.VMEM((16,), jnp.float32),
            pltpu.VMEM((16 * 64,), jnp.float32),
            pltpu.VMEM((64,), jnp.float32),
            pltpu.VMEM_SHARED((16 * _S,), jnp.float32),
        ],
    )
    def seg(y_hbm, batch_hbm, shift_hbm, out_hbm,
            y_v, b_v, acc_v, shift_v, red_v, out_v, shared):
        s = lax.axis_index("s")

        zf = jnp.zeros((16,), jnp.float32)
        for i in range(_S // 16):
            acc_v[pl.ds(i * 16, 16)] = zf

        pltpu.sync_copy(shift_hbm, shift_v)
        own_blk = _WBLK * s - jnp.maximum(0, s - 10)
        nblk = jnp.where(s < 10, _WBLK, _WBLK - 1)
        dma_blk = jnp.minimum(own_blk, _NB16 - _WBLK)
        base = dma_blk * 16
        pltpu.sync_copy(y_hbm.at[pl.ds(base, _CHUNK)], y_v)
        pltpu.sync_copy(batch_hbm.at[pl.ds(base, _CHUNK)], b_v)

        startb = own_blk - dma_blk

        def body(b, carry):
            off = b * 16
            plsc.addupdate_scatter(
                acc_v, [b_v[pl.ds(off, 16)]], y_v[pl.ds(off, 16)])
            return carry

        lax.fori_loop(startb, startb + nblk, body, 0)

        pltpu.sync_copy(acc_v, shared.at[pl.ds(s * _S, _S)])
        plsc.subcore_barrier()

        col = s * 64
        for t in range(16):
            pltpu.sync_copy(shared.at[pl.ds(t * _S + col, 64)],
                            red_v.at[pl.ds(t * 64, 64)])
        sh = shift_v[...]
        for j in range(4):
            v = red_v[pl.ds(j * 16, 16)] + sh
            for t in range(1, 16):
                v = v + red_v[pl.ds(t * 64 + j * 16, 16)]
            out_v[pl.ds(j * 16, 16)] = v
        pltpu.sync_copy(out_v, out_hbm.at[pl.ds(col, 64)])

    return seg


def kernel(h, batch, W0, b0, W1, b1, energy_scaling_coeff,
           energy_shifting_coeff):
    y = _node_energies(h, W0, b0, W1, b1, energy_scaling_coeff)
    shift_vec = jnp.full((16,), energy_shifting_coeff, dtype=jnp.float32)
    return _make_segment_sum()(y, batch, shift_vec)


# speedup vs baseline: 1.0503x; 1.0503x over previous
import functools

import jax
import jax.numpy as jnp
from jax import lax
from jax.experimental import pallas as pl
from jax.experimental.pallas import tpu as pltpu
from jax.experimental.pallas import tpu_sc as plsc

_N = 100000
_D = 128
_H = 64
_S = 1024

_BLK = 20000
_NB16 = _N // 16
_WBLK = 196
_CHUNK = _WBLK * 16


def _mlp_body(h_ref, w0_ref, b0_ref, w1_ref, b1_ref, sc_ref, y_ref):
    wf = jnp.dot(w0_ref[...], w1_ref[...], preferred_element_type=jnp.float32)
    wf = wf * sc_ref[...]
    c = (jnp.dot(b0_ref[...], w1_ref[...], preferred_element_type=jnp.float32)
         + b1_ref[...]) * sc_ref[...]
    y_ref[...] = jnp.dot(h_ref[...], wf,
                         preferred_element_type=jnp.float32) + c


def _node_energies(h, W0, b0, W1, b1, scale):
    return pl.pallas_call(
        _mlp_body,
        grid=(_N // _BLK,),
        in_specs=[
            pl.BlockSpec((_BLK, _D), lambda i: (i, 0)),
            pl.BlockSpec((_D, _H), lambda i: (0, 0)),
            pl.BlockSpec((1, _H), lambda i: (0, 0)),
            pl.BlockSpec((_H, 1), lambda i: (0, 0)),
            pl.BlockSpec((1, 1), lambda i: (0, 0)),
            pl.BlockSpec((1, 1), lambda i: (0, 0)),
        ],
        out_specs=pl.BlockSpec((_BLK, 1), lambda i: (i, 0)),
        out_shape=jax.ShapeDtypeStruct((_N, 1), jnp.float32),
    )(h, W0, b0.reshape(1, _H), W1, b1.reshape(1, 1), scale.reshape(1, 1))


def _make_segment_sum():
    mesh = plsc.VectorSubcoreMesh(core_axis_name="c", subcore_axis_name="s")

    @functools.partial(
        pl.kernel,
        mesh=mesh,
        compiler_params=pltpu.CompilerParams(needs_layout_passes=False),
        out_type=jax.ShapeDtypeStruct((2 * _S,), jnp.float32),
        scratch_types=[
            pltpu.VMEM((_CHUNK,), jnp.float32),
            pltpu.VMEM((_CHUNK,), jnp.int32),
            pltpu.VMEM((_S,), jnp.float32),
            pltpu.VMEM((16 * 64,), jnp.float32),
            pltpu.VMEM((64,), jnp.float32),
            pltpu.VMEM_SHARED((16 * _S,), jnp.float32),
        ],
    )
    def seg(y_hbm, batch_hbm, out_hbm, y_v, b_v, acc_v, red_v, out_v, shared):
        s = lax.axis_index("s")
        c = lax.axis_index("c")
        w = s * 2 + c

        zf = jnp.zeros((16,), jnp.float32)

        def zero(i, carry):
            acc_v[pl.ds(i * 16, 16)] = zf
            return carry

        lax.fori_loop(0, _S // 16, zero, 0)

        own_blk = _WBLK * w - jnp.maximum(0, w - 10)
        nblk = jnp.where(w < 10, _WBLK, _WBLK - 1)
        dma_blk = jnp.minimum(own_blk, _NB16 - _WBLK)
        base = dma_blk * 16
        pltpu.sync_copy(y_hbm.at[pl.ds(base, _CHUNK)], y_v)
        pltpu.sync_copy(batch_hbm.at[pl.ds(base, _CHUNK)], b_v)

        startb = own_blk - dma_blk
        endb = startb + nblk

        def body(i, carry):
            for k in range(4):
                b = i * 4 + k
                off = b * 16
                valid = jnp.logical_and(b >= startb, b < endb)
                mask = jnp.full((16,), True, dtype=jnp.bool_) & valid
                plsc.addupdate_scatter(
                    acc_v, [b_v[pl.ds(off, 16)]], y_v[pl.ds(off, 16)],
                    mask=mask)
            return carry

        lax.fori_loop(0, _WBLK // 4, body, 0)

        pltpu.sync_copy(acc_v, shared.at[pl.ds(s * _S, _S)])
        plsc.subcore_barrier()

        col = s * 64
        for t in range(16):
            pltpu.sync_copy(shared.at[pl.ds(t * _S + col, 64)],
                            red_v.at[pl.ds(t * 64, 64)])
        for j in range(4):
            v = red_v[pl.ds(j * 16, 16)]
            for t in range(1, 16):
                v = v + red_v[pl.ds(t * 64 + j * 16, 16)]
            out_v[pl.ds(j * 16, 16)] = v
        pltpu.sync_copy(out_v, out_hbm.at[pl.ds(c * _S + col, 64)])

    return seg


def _comb_body(p_ref, sh_ref, o_ref):
    o_ref[...] = p_ref[0:1, :] + p_ref[1:2, :] + sh_ref[...]


def _combine(partials, shift):
    return pl.pallas_call(
        _comb_body,
        out_shape=jax.ShapeDtypeStruct((1, _S), jnp.float32),
    )(partials, shift.reshape(1, 1))


def kernel(h, batch, W0, b0, W1, b1, energy_scaling_coeff,
           energy_shifting_coeff):
    y = _node_energies(h, W0, b0, W1, b1, energy_scaling_coeff).reshape(_N)
    partials = _make_segment_sum()(y, batch).reshape(2, _S)
    return _combine(partials, energy_shifting_coeff).reshape(_S)


# speedup vs baseline: 1.0751x; 1.0236x over previous
import functools

import jax
import jax.numpy as jnp
from jax import lax
from jax.experimental import pallas as pl
from jax.experimental.pallas import tpu as pltpu
from jax.experimental.pallas import tpu_sc as plsc

_N = 100000
_D = 128
_H = 64
_S = 1024

_BLK = 20000
_NB16 = _N // 16
_WBLK = 196
_CHUNK = _WBLK * 16


def _mlp_body(h_ref, w0_ref, b0_ref, w1_ref, b1_ref, sc_ref, y_ref):
    wf = jnp.dot(w0_ref[...], w1_ref[...], preferred_element_type=jnp.float32)
    wf = wf * sc_ref[...]
    c = (jnp.dot(b0_ref[...], w1_ref[...], preferred_element_type=jnp.float32)
         + b1_ref[...]) * sc_ref[...]
    y_ref[...] = jnp.dot(h_ref[...], wf,
                         preferred_element_type=jnp.float32) + c


def _node_energies(h, W0, b0, W1, b1, scale):
    return pl.pallas_call(
        _mlp_body,
        grid=(_N // _BLK,),
        in_specs=[
            pl.BlockSpec((_BLK, _D), lambda i: (i, 0)),
            pl.BlockSpec((_D, _H), lambda i: (0, 0)),
            pl.BlockSpec((1, _H), lambda i: (0, 0)),
            pl.BlockSpec((_H, 1), lambda i: (0, 0)),
            pl.BlockSpec((1, 1), lambda i: (0, 0)),
            pl.BlockSpec((1, 1), lambda i: (0, 0)),
        ],
        out_specs=pl.BlockSpec((_BLK, 1), lambda i: (i, 0)),
        out_shape=jax.ShapeDtypeStruct((_N, 1), jnp.float32),
    )(h, W0, b0.reshape(1, _H), W1, b1.reshape(1, 1), scale.reshape(1, 1))


def _make_segment_sum():
    mesh = plsc.VectorSubcoreMesh(core_axis_name="c", subcore_axis_name="s")

    @functools.partial(
        pl.kernel,
        mesh=mesh,
        compiler_params=pltpu.CompilerParams(needs_layout_passes=False),
        out_type=jax.ShapeDtypeStruct((2 * _S,), jnp.float32),
        scratch_types=[
            pltpu.VMEM((_CHUNK,), jnp.float32),
            pltpu.VMEM((_CHUNK,), jnp.int32),
            pltpu.VMEM((2 * _S,), jnp.float32),
            pltpu.VMEM((32 * 64,), jnp.float32),
            pltpu.VMEM((64,), jnp.float32),
            pltpu.VMEM_SHARED((16 * 2 * _S,), jnp.float32),
            pltpu.SemaphoreType.DMA,
        ],
    )
    def seg(y_hbm, batch_hbm, out_hbm,
            y_v, b_v, acc_v, red_v, out_v, shared, sem):
        s = lax.axis_index("s")
        c = lax.axis_index("c")
        w = s * 2 + c

        zf = jnp.zeros((16,), jnp.float32)

        def zero(i, carry):
            acc_v[pl.ds(i * 16, 16)] = zf
            return carry

        lax.fori_loop(0, 2 * _S // 16, zero, 0)

        own_blk = _WBLK * w - jnp.maximum(0, w - 10)
        nblk = jnp.where(w < 10, _WBLK, _WBLK - 1)
        dma_blk = jnp.minimum(own_blk, _NB16 - _WBLK)
        base = dma_blk * 16
        pltpu.sync_copy(y_hbm.at[pl.ds(base, _CHUNK)], y_v)
        pltpu.sync_copy(batch_hbm.at[pl.ds(base, _CHUNK)], b_v)

        startb = own_blk - dma_blk
        endb = startb + nblk
        bank = (lax.iota(jnp.int32, 16) & 1) * _S

        def body(i, carry):
            for k in range(4):
                b = i * 4 + k
                off = b * 16
                valid = jnp.logical_and(b >= startb, b < endb)
                mask = jnp.full((16,), True, dtype=jnp.bool_) & valid
                plsc.addupdate_scatter(
                    acc_v, [b_v[pl.ds(off, 16)] + bank], y_v[pl.ds(off, 16)],
                    mask=mask)
            return carry

        lax.fori_loop(0, _WBLK // 4, body, 0)

        pltpu.sync_copy(acc_v, shared.at[pl.ds(s * 2 * _S, 2 * _S)])
        plsc.subcore_barrier()

        col = s * 64
        copies = []
        for t in range(16):
            copies.append(pltpu.async_copy(
                shared.at[pl.ds(t * 2 * _S + col, 64)],
                red_v.at[pl.ds(2 * t * 64, 64)], sem))
            copies.append(pltpu.async_copy(
                shared.at[pl.ds(t * 2 * _S + _S + col, 64)],
                red_v.at[pl.ds((2 * t + 1) * 64, 64)], sem))
        for cp in copies:
            cp.wait()
        for j in range(4):
            v = red_v[pl.ds(j * 16, 16)]
            for r in range(1, 32):
                v = v + red_v[pl.ds(r * 64 + j * 16, 16)]
            out_v[pl.ds(j * 16, 16)] = v
        pltpu.sync_copy(out_v, out_hbm.at[pl.ds(c * _S + col, 64)])

    return seg


def _comb_body(p_ref, sh_ref, o_ref):
    o_ref[...] = p_ref[0:1, :] + p_ref[1:2, :] + sh_ref[...]


def _combine(partials, shift):
    return pl.pallas_call(
        _comb_body,
        out_shape=jax.ShapeDtypeStruct((1, _S), jnp.float32),
    )(partials, shift.reshape(1, 1))


def kernel(h, batch, W0, b0, W1, b1, energy_scaling_coeff,
           energy_shifting_coeff):
    y = _node_energies(h, W0, b0, W1, b1, energy_scaling_coeff).reshape(_N)
    partials = _make_segment_sum()(y, batch).reshape(2, _S)
    return _combine(partials, energy_shifting_coeff).reshape(_S)
